# baseline (speedup 1.0000x reference)
import functools

import jax
import jax.numpy as jnp
from jax import lax
from jax.experimental import pallas as pl
from jax.experimental.pallas import tpu as pltpu
from jax.experimental.pallas import tpu_sc as plsc

VOCAB = 1000000
EMBD = 64
B = 4096
L = 50

NW = 32
N = B * L
CHUNK = 128
NCHUNK = N // (NW * CHUNK)


def _emb_body(x_hbm, table_hbm, out_hbm, idx_v, rows_v, gsem, wsem):
    nc = 2
    wid = lax.axis_index("s") * nc + lax.axis_index("c")

    pltpu.sync_copy(x_hbm.at[wid], idx_v)

    def gather(j, b):
        pltpu.async_copy(table_hbm.at[idx_v.at[j]], rows_v.at[b], gsem.at[b])

    def gather_wait(j, b):
        pltpu.make_async_copy(table_hbm.at[idx_v.at[j]], rows_v.at[b],
                              gsem.at[b]).wait()

    def write(j, b):
        pltpu.async_copy(rows_v.at[b], out_hbm.at[wid, j], wsem.at[b])

    def write_wait(j, b):
        pltpu.make_async_copy(rows_v.at[b], out_hbm.at[wid, j],
                              wsem.at[b]).wait()

    gather(0, 0)

    @pl.loop(0, NCHUNK // 2)
    def _(p):
        for b in range(2):
            j = 2 * p + b
            bn = 1 - b
            gather_wait(j, b)

            def overlap(j=j, b=b, bn=bn):
                def drain(j=j, bn=bn):
                    write_wait(j - 1, bn)

                if b == 0:
                    pl.when(j >= 1)(drain)
                else:
                    drain()
                gather(j + 1, bn)

            if b == 0:
                overlap()
            else:
                pl.when(j + 1 < NCHUNK)(overlap)

            write(j, b)

    write_wait(NCHUNK - 2, 0)
    write_wait(NCHUNK - 1, 1)


@jax.jit
def _emb(x3, table):
    mesh = plsc.VectorSubcoreMesh(core_axis_name="c", subcore_axis_name="s")
    f = pl.kernel(
        _emb_body,
        out_type=jax.ShapeDtypeStruct((NW, NCHUNK, CHUNK, EMBD), jnp.float32),
        mesh=mesh,
        compiler_params=pltpu.CompilerParams(use_tc_tiling_on_sc=False),
        scratch_types=[
            pltpu.VMEM((NCHUNK, CHUNK), jnp.int32),
            pltpu.VMEM((2, CHUNK, EMBD), jnp.float32),
            pltpu.SemaphoreType.DMA((2,)),
            pltpu.SemaphoreType.DMA((2,)),
        ],
    )
    return f(x3, table)


def kernel(x, table):
    x3 = x.reshape(NW, NCHUNK, CHUNK).astype(jnp.int32)
    out = _emb(x3, table)
    return out.reshape(B, L, EMBD)


# speedup vs baseline: 1.0206x; 1.0206x over previous
import jax
import jax.numpy as jnp
from jax import lax
from jax.experimental import pallas as pl
from jax.experimental.pallas import tpu as pltpu
from jax.experimental.pallas import tpu_sc as plsc

VOCAB = 1000000
EMBD = 64
B = 4096
L = 50

NW = 32
BPW = B // NW
GB = 4
NG = BPW // GB


def _emb_body(x_hbm, table_hbm, out_hbm, idx_v, rows_v, gsem, wsem):
    nc = 2
    wid = lax.axis_index("s") * nc + lax.axis_index("c")
    b0 = wid * BPW

    pltpu.sync_copy(x_hbm.at[pl.ds(b0, BPW)], idx_v)

    def gather(g, k, p):
        pltpu.async_copy(table_hbm.at[idx_v.at[g * GB + k]],
                         rows_v.at[p, k], gsem.at[p, k])

    def gather_wait(g, k, p):
        pltpu.make_async_copy(table_hbm.at[idx_v.at[g * GB + k]],
                              rows_v.at[p, k], gsem.at[p, k]).wait()

    def write(g, p):
        pltpu.async_copy(rows_v.at[p], out_hbm.at[pl.ds(b0 + g * GB, GB)],
                         wsem.at[p])

    def write_wait(g, p):
        pltpu.make_async_copy(rows_v.at[p], out_hbm.at[pl.ds(b0 + g * GB, GB)],
                              wsem.at[p]).wait()

    for k in range(GB):
        gather(0, k, 0)

    @pl.loop(0, NG // 2)
    def _(gg):
        for p in range(2):
            g = 2 * gg + p

            for k in range(GB):
                gather_wait(g, k, p)
            write(g, p)

            def prefetch(g=g, p=p):
                def drain(g=g, p=p):
                    write_wait(g - 1, 1 - p)

                if p == 1:
                    drain()
                else:
                    pl.when(g >= 1)(drain)
                for k in range(GB):
                    gather(g + 1, k, 1 - p)

            if p == 0:
                prefetch()
            else:
                pl.when(gg < NG // 2 - 1)(prefetch)

    write_wait(NG - 2, 0)
    write_wait(NG - 1, 1)


@jax.jit
def _emb(x, table):
    mesh = plsc.VectorSubcoreMesh(core_axis_name="c", subcore_axis_name="s")
    f = pl.kernel(
        _emb_body,
        out_type=jax.ShapeDtypeStruct((B, L, EMBD), jnp.float32),
        mesh=mesh,
        compiler_params=pltpu.CompilerParams(use_tc_tiling_on_sc=False),
        scratch_types=[
            pltpu.VMEM((BPW, L), jnp.int32),
            pltpu.VMEM((2, GB, L, EMBD), jnp.float32),
            pltpu.SemaphoreType.DMA((2, GB)),
            pltpu.SemaphoreType.DMA((2,)),
        ],
    )
    return f(x, table)


def kernel(x, table):
    return _emb(x.astype(jnp.int32), table)
